# baseline (speedup 1.0000x reference)
import functools

import jax
import jax.numpy as jnp
from jax import lax
from jax.experimental import pallas as pl
from jax.experimental.pallas import tpu as pltpu
from jax.experimental.pallas import tpu_sc as plsc


def _combine_tables(depth_embed, width_embed):
    VD, D = depth_embed.shape
    VW, _ = width_embed.shape

    def body(d_ref, w_ref, o_ref):
        d = d_ref[...]
        w = w_ref[...]
        o_ref[...] = d[:, None, :] + w[None, :, :]

    out3 = pl.pallas_call(
        body,
        out_shape=jax.ShapeDtypeStruct((VD, VW, D), jnp.float32),
    )(depth_embed, width_embed)
    return out3.reshape(VD * VW, D)


@functools.partial(jax.jit, static_argnums=(3, 4, 5))
def _sc_gather(d_idx, w_idx, comb, d_model, vd_max, vw_max):
    seq = d_idx.shape[0]
    info = plsc.get_sparse_core_info()
    NC, NS = info.num_cores, info.num_subcores
    NW = NC * NS
    bpw = seq // NW
    C = 32
    nch = bpw // C
    per = C // 16

    mesh = plsc.VectorSubcoreMesh(core_axis_name="c", subcore_axis_name="s")

    @functools.partial(
        pl.kernel,
        mesh=mesh,
        out_type=jax.ShapeDtypeStruct((seq, d_model), jnp.float32),
        scratch_types=[
            pltpu.VMEM((bpw,), jnp.int32),
            pltpu.VMEM((bpw,), jnp.int32),
            pltpu.VMEM((nch, C), jnp.int32),
            pltpu.VMEM((C, d_model), jnp.float32),
            pltpu.VMEM((C, d_model), jnp.float32),
            pltpu.SemaphoreType.DMA,
            pltpu.SemaphoreType.DMA,
            pltpu.SemaphoreType.DMA,
            pltpu.SemaphoreType.DMA,
        ],
    )
    def k(d_hbm, w_hbm, comb_hbm, out_hbm, dv, wv, cidx,
          buf0, buf1, gsem0, gsem1, ssem0, ssem1):
        wid = lax.axis_index("s") * NC + lax.axis_index("c")
        base = wid * bpw
        bufs = (buf0, buf1)
        gsems = (gsem0, gsem1)
        ssems = (ssem0, ssem1)
        pltpu.sync_copy(d_hbm.at[pl.ds(base, bpw)], dv)
        pltpu.sync_copy(w_hbm.at[pl.ds(base, bpw)], wv)
        for kk in range(bpw // 16):
            d = dv[pl.ds(kk * 16, 16)]
            w = wv[pl.ds(kk * 16, 16)]
            d = jnp.minimum(jnp.maximum(d, 0), vd_max)
            w = jnp.minimum(jnp.maximum(w, 0), vw_max)
            cidx[kk // per, pl.ds((kk % per) * 16, 16)] = d * (vw_max + 1) + w

        def gather(j, b):
            return pltpu.async_copy(comb_hbm.at[cidx.at[j]], bufs[b], gsems[b])

        def scatter(j, b):
            return pltpu.async_copy(
                bufs[b], out_hbm.at[pl.ds(base + j * C, C)], ssems[b])

        gd = [None, None]
        sd = [None, None]
        gd[0] = gather(0, 0)
        for j in range(nch):
            cur = j % 2
            oth = 1 - cur
            gd[cur].wait()
            if j + 1 < nch:
                if j >= 1:
                    sd[oth].wait()
                gd[oth] = gather(j + 1, oth)
            sd[cur] = scatter(j, cur)
        sd[0].wait()
        sd[1].wait()

    return k(d_idx, w_idx, comb)


def kernel(depth_indices, width_indices, depth_embed, width_embed):
    seq = depth_indices.shape[0]
    D = depth_embed.shape[1]
    comb = _combine_tables(depth_embed, width_embed)
    d = depth_indices.reshape(seq).astype(jnp.int32)
    w = width_indices.reshape(seq).astype(jnp.int32)
    out = _sc_gather(d, w, comb, D,
                     depth_embed.shape[0] - 1, width_embed.shape[0] - 1)
    return out.reshape(seq, 1, D)


# speedup vs baseline: 1.5041x; 1.5041x over previous
import functools

import jax
import jax.numpy as jnp
from jax import lax
from jax.experimental import pallas as pl
from jax.experimental.pallas import tpu as pltpu
from jax.experimental.pallas import tpu_sc as plsc


def _combine_tables(depth_embed, width_embed):
    VD, D = depth_embed.shape
    VW, _ = width_embed.shape

    def body(d_ref, w_ref, o_ref):
        d = d_ref[...]
        w = w_ref[...]
        o_ref[...] = d[:, None, :] + w[None, :, :]

    out3 = pl.pallas_call(
        body,
        out_shape=jax.ShapeDtypeStruct((VD, VW, D), jnp.float32),
    )(depth_embed, width_embed)
    return out3.reshape(VD * VW, D)


@functools.partial(jax.jit, static_argnums=(3, 4, 5))
def _sc_gather(d_idx, w_idx, comb, d_model, vd_max, vw_max):
    seq = d_idx.shape[0]
    info = plsc.get_sparse_core_info()
    NC, NS = info.num_cores, info.num_subcores
    NW = NC * NS
    bpw = seq // NW
    C = 32
    nch = bpw // C
    per = C // 16

    mesh = plsc.VectorSubcoreMesh(core_axis_name="c", subcore_axis_name="s")

    @functools.partial(
        pl.kernel,
        mesh=mesh,
        out_type=jax.ShapeDtypeStruct((seq, 1, d_model), jnp.float32),
        scratch_types=[
            pltpu.VMEM((bpw,), jnp.int32),
            pltpu.VMEM((bpw,), jnp.int32),
            pltpu.VMEM((nch, C), jnp.int32),
            pltpu.VMEM((C, d_model), jnp.float32),
            pltpu.VMEM((C, d_model), jnp.float32),
            pltpu.SemaphoreType.DMA,
            pltpu.SemaphoreType.DMA,
            pltpu.SemaphoreType.DMA,
            pltpu.SemaphoreType.DMA,
        ],
    )
    def k(d_hbm, w_hbm, comb_hbm, out_hbm, dv, wv, cidx,
          buf0, buf1, gsem0, gsem1, ssem0, ssem1):
        wid = lax.axis_index("s") * NC + lax.axis_index("c")
        base = wid * bpw
        bufs = (buf0, buf1)
        gsems = (gsem0, gsem1)
        ssems = (ssem0, ssem1)
        pltpu.sync_copy(d_hbm.at[pl.ds(base, bpw)], dv)
        pltpu.sync_copy(w_hbm.at[pl.ds(base, bpw)], wv)
        for kk in range(bpw // 16):
            d = dv[pl.ds(kk * 16, 16)]
            w = wv[pl.ds(kk * 16, 16)]
            d = jnp.minimum(jnp.maximum(d, 0), vd_max)
            w = jnp.minimum(jnp.maximum(w, 0), vw_max)
            cidx[kk // per, pl.ds((kk % per) * 16, 16)] = d * (vw_max + 1) + w

        def gather(j, b):
            return pltpu.async_copy(comb_hbm.at[cidx.at[j]], bufs[b], gsems[b])

        def scatter(j, b):
            return pltpu.async_copy(
                bufs[b], out_hbm.at[pl.ds(base + j * C, C), 0], ssems[b])

        gd = [None, None]
        sd = [None, None]
        gd[0] = gather(0, 0)
        for j in range(nch):
            cur = j % 2
            oth = 1 - cur
            gd[cur].wait()
            if j + 1 < nch:
                if j >= 1:
                    sd[oth].wait()
                gd[oth] = gather(j + 1, oth)
            sd[cur] = scatter(j, cur)
        sd[0].wait()
        sd[1].wait()

    return k(d_idx, w_idx, comb)


def kernel(depth_indices, width_indices, depth_embed, width_embed):
    seq = depth_indices.shape[0]
    D = depth_embed.shape[1]
    comb = _combine_tables(depth_embed, width_embed)
    d = depth_indices.reshape(seq).astype(jnp.int32)
    w = width_indices.reshape(seq).astype(jnp.int32)
    return _sc_gather(d, w, comb, D,
                      depth_embed.shape[0] - 1, width_embed.shape[0] - 1)
